# baseline (speedup 1.0000x reference)
import functools

import jax
import jax.numpy as jnp
from jax import lax
from jax.experimental import pallas as pl
from jax.experimental.pallas import tpu as pltpu
from jax.experimental.pallas import tpu_sc as plsc

B = 16384
F = 26
V = 100000
D = 32
NE = F * D

NC = 2
NS = 16
NW = NC * NS
CHUNK = 128
RPW = (B * F) // NW
CPW = RPW // CHUNK


def _sc_gather(idx2d, tab_flat, nf):
    rpw = (B * nf) // NW
    cpw = rpw // CHUNK
    mesh = plsc.VectorSubcoreMesh(core_axis_name="c", subcore_axis_name="s")

    @functools.partial(
        pl.kernel,
        mesh=mesh,
        compiler_params=pltpu.CompilerParams(use_tc_tiling_on_sc=False),
        out_type=jax.ShapeDtypeStruct((B * nf, D), jnp.float32),
        scratch_types=[
            pltpu.VMEM((cpw, CHUNK), jnp.int32),
            pltpu.VMEM((2, CHUNK, D), jnp.float32),
            pltpu.SemaphoreType.DMA,
            pltpu.SemaphoreType.DMA,
            pltpu.SemaphoreType.DMA,
            pltpu.SemaphoreType.DMA,
        ],
    )
    def gather_kernel(idx_hbm, tab_hbm, out_hbm, idx_v, rows_v, g0, g1, s0, s1):
        wid = lax.axis_index("s") * NC + lax.axis_index("c")
        pltpu.sync_copy(idx_hbm.at[pl.ds(wid * cpw, cpw)], idx_v)
        base = wid * rpw
        npair = cpw // 2

        def gather(c, b, sem):
            pltpu.async_copy(tab_hbm.at[idx_v.at[c]], rows_v.at[b], sem)

        def wait_gather(c, b, sem):
            pltpu.make_async_copy(
                tab_hbm.at[idx_v.at[c]], rows_v.at[b], sem
            ).wait()

        def write(c, b, sem):
            pltpu.async_copy(
                rows_v.at[b], out_hbm.at[pl.ds(base + c * CHUNK, CHUNK)], sem
            )

        def wait_write(c, b, sem):
            pltpu.make_async_copy(
                rows_v.at[b], out_hbm.at[pl.ds(base + c * CHUNK, CHUNK)], sem
            ).wait()

        gather(0, 0, g0)
        gather(1, 1, g1)

        def body(j, carry):
            c0 = 2 * j
            c1 = c0 + 1
            wait_gather(c0, 0, g0)
            write(c0, 0, s0)
            wait_gather(c1, 1, g1)
            write(c1, 1, s1)
            wait_write(c0, 0, s0)

            @pl.when(j < npair - 1)
            def _():
                gather(c0 + 2, 0, g0)

            wait_write(c1, 1, s1)

            @pl.when(j < npair - 1)
            def _():
                gather(c1 + 2, 1, g1)

            return carry

        lax.fori_loop(0, npair, body, 0)

    return gather_kernel(idx2d, tab_flat)


NBLK = 16
BLK = B // NBLK


def _stage1_body(e0_ref, e1_ref, w1a_ref, w1b_ref, b1_ref, g_ref, bt_ref,
                 h1_ref, st_ref, s_ref, q_ref):
    i = pl.program_id(0)
    h = jnp.dot(e0_ref[...], w1a_ref[...], preferred_element_type=jnp.float32)
    h = h + jnp.dot(e1_ref[...], w1b_ref[...],
                    preferred_element_type=jnp.float32)
    h = jnp.maximum(h + b1_ref[...], 0.0)
    h1_ref[...] = h
    ps = jnp.sum(h, axis=0, keepdims=True)
    pq = jnp.sum(h * h, axis=0, keepdims=True)

    @pl.when(i == 0)
    def _():
        s_ref[...] = ps
        q_ref[...] = pq

    @pl.when(i > 0)
    def _():
        s_ref[...] += ps
        q_ref[...] += pq

    @pl.when(i == NBLK - 1)
    def _():
        mu = s_ref[...] * (1.0 / B)
        var = q_ref[...] * (1.0 / B) - mu * mu
        scale = g_ref[...] * lax.rsqrt(jnp.maximum(var, 0.0) + 1e-5)
        st_ref[0:1, :] = scale
        st_ref[1:2, :] = bt_ref[...] - mu * scale


def _stage2_body(h1_ref, st_ref, w2_ref, b2_ref, w3_ref, b3_ref, out_ref):
    h = h1_ref[...] * st_ref[0:1, :] + st_ref[1:2, :]
    h = jnp.dot(h, w2_ref[...], preferred_element_type=jnp.float32)
    h = jnp.maximum(h + b2_ref[...], 0.0)
    out_ref[...] = (
        jnp.dot(h, w3_ref[...], preferred_element_type=jnp.float32)
        + b3_ref[...]
    )


NEH = NE // 2


def _mlp(emb0, emb1, W1, b1, gamma, beta, W2, b2, W3, b3):
    h1, stats = pl.pallas_call(
        _stage1_body,
        grid=(NBLK,),
        in_specs=[
            pl.BlockSpec((BLK, NEH), lambda i: (i, 0)),
            pl.BlockSpec((BLK, NEH), lambda i: (i, 0)),
            pl.BlockSpec((NEH, 128), lambda i: (0, 0)),
            pl.BlockSpec((NEH, 128), lambda i: (0, 0)),
            pl.BlockSpec((1, 128), lambda i: (0, 0)),
            pl.BlockSpec((1, 128), lambda i: (0, 0)),
            pl.BlockSpec((1, 128), lambda i: (0, 0)),
        ],
        out_specs=[
            pl.BlockSpec((BLK, 128), lambda i: (i, 0)),
            pl.BlockSpec((2, 128), lambda i: (0, 0)),
        ],
        out_shape=[
            jax.ShapeDtypeStruct((B, 128), jnp.float32),
            jax.ShapeDtypeStruct((2, 128), jnp.float32),
        ],
        scratch_shapes=[
            pltpu.VMEM((1, 128), jnp.float32),
            pltpu.VMEM((1, 128), jnp.float32),
        ],
    )(emb0, emb1, W1[:NEH], W1[NEH:], b1.reshape(1, 128),
      gamma.reshape(1, 128), beta.reshape(1, 128))

    out = pl.pallas_call(
        _stage2_body,
        grid=(NBLK,),
        in_specs=[
            pl.BlockSpec((BLK, 128), lambda i: (i, 0)),
            pl.BlockSpec((2, 128), lambda i: (0, 0)),
            pl.BlockSpec((128, 64), lambda i: (0, 0)),
            pl.BlockSpec((1, 64), lambda i: (0, 0)),
            pl.BlockSpec((64, 1), lambda i: (0, 0)),
            pl.BlockSpec((1, 1), lambda i: (0, 0)),
        ],
        out_specs=pl.BlockSpec((BLK, 1), lambda i: (i, 0)),
        out_shape=jax.ShapeDtypeStruct((B, 1), jnp.float32),
    )(h1, stats, W2, b2.reshape(1, 64), W3, b3.reshape(1, 1))
    return out


FH = F // 2


def kernel(x, tables, W1, b1, gamma, beta, W2, b2, W3, b3):
    offs = (jnp.arange(FH, dtype=jnp.int32) * V)[None, :]
    idx0 = (x[:, :FH] + offs).reshape((B * FH) // CHUNK, CHUNK)
    idx1 = (x[:, FH:] + offs).reshape((B * FH) // CHUNK, CHUNK)
    emb0 = _sc_gather(idx0, tables[:FH].reshape(FH * V, D), FH).reshape(B, NEH)
    emb1 = _sc_gather(idx1, tables[FH:].reshape(FH * V, D), FH).reshape(B, NEH)
    return _mlp(emb0, emb1, W1, b1, gamma, beta, W2, b2, W3, b3)


# speedup vs baseline: 1.4757x; 1.4757x over previous
import functools

import jax
import jax.numpy as jnp
from jax import lax
from jax.experimental import pallas as pl
from jax.experimental.pallas import tpu as pltpu
from jax.experimental.pallas import tpu_sc as plsc

B = 16384
F = 26
V = 100000
D = 32
NE = F * D

NC = 2
NS = 16
NW = NC * NS
CHUNK = 128
RPW = (B * F) // NW
CPW = RPW // CHUNK


def _sc_gather(idx2d, tab_flat):
    mesh = plsc.VectorSubcoreMesh(core_axis_name="c", subcore_axis_name="s")

    @functools.partial(
        pl.kernel,
        mesh=mesh,
        compiler_params=pltpu.CompilerParams(use_tc_tiling_on_sc=False),
        out_type=jax.ShapeDtypeStruct((B * F, D), jnp.float32),
        scratch_types=[
            pltpu.VMEM((CPW, CHUNK), jnp.int32),
            pltpu.VMEM((2, CHUNK, D), jnp.float32),
            pltpu.SemaphoreType.DMA,
            pltpu.SemaphoreType.DMA,
            pltpu.SemaphoreType.DMA,
            pltpu.SemaphoreType.DMA,
        ],
    )
    def gather_kernel(idx_hbm, tab_hbm, out_hbm, idx_v, rows_v, g0, g1, s0, s1):
        wid = lax.axis_index("s") * NC + lax.axis_index("c")
        pltpu.sync_copy(idx_hbm.at[pl.ds(wid * CPW, CPW)], idx_v)
        base = wid * RPW
        npair = CPW // 2

        def gather(c, b, sem):
            pltpu.async_copy(tab_hbm.at[idx_v.at[c]], rows_v.at[b], sem)

        def wait_gather(c, b, sem):
            pltpu.make_async_copy(
                tab_hbm.at[idx_v.at[c]], rows_v.at[b], sem
            ).wait()

        def write(c, b, sem):
            pltpu.async_copy(
                rows_v.at[b], out_hbm.at[pl.ds(base + c * CHUNK, CHUNK)], sem
            )

        def wait_write(c, b, sem):
            pltpu.make_async_copy(
                rows_v.at[b], out_hbm.at[pl.ds(base + c * CHUNK, CHUNK)], sem
            ).wait()

        gather(0, 0, g0)
        gather(1, 1, g1)

        def body(j, carry):
            c0 = 2 * j
            c1 = c0 + 1
            wait_gather(c0, 0, g0)
            write(c0, 0, s0)
            wait_gather(c1, 1, g1)
            write(c1, 1, s1)
            wait_write(c0, 0, s0)

            @pl.when(j < npair - 1)
            def _():
                gather(c0 + 2, 0, g0)

            wait_write(c1, 1, s1)

            @pl.when(j < npair - 1)
            def _():
                gather(c1 + 2, 1, g1)

            return carry

        lax.fori_loop(0, npair, body, 0)

    return gather_kernel(idx2d, tab_flat)


NBLK = 16
BLK = B // NBLK


def _stage1_body(emb_ref, w1_ref, b1_ref, g_ref, bt_ref, h1_ref, st_ref,
                 s_ref, q_ref):
    i = pl.program_id(0)
    h = jnp.dot(emb_ref[...], w1_ref[...], preferred_element_type=jnp.float32)
    h = jnp.maximum(h + b1_ref[...], 0.0)
    h1_ref[...] = h
    ps = jnp.sum(h, axis=0, keepdims=True)
    pq = jnp.sum(h * h, axis=0, keepdims=True)

    @pl.when(i == 0)
    def _():
        s_ref[...] = ps
        q_ref[...] = pq

    @pl.when(i > 0)
    def _():
        s_ref[...] += ps
        q_ref[...] += pq

    @pl.when(i == NBLK - 1)
    def _():
        mu = s_ref[...] * (1.0 / B)
        var = q_ref[...] * (1.0 / B) - mu * mu
        scale = g_ref[...] * lax.rsqrt(jnp.maximum(var, 0.0) + 1e-5)
        st_ref[0:1, :] = scale
        st_ref[1:2, :] = bt_ref[...] - mu * scale


def _stage2_body(h1_ref, st_ref, w2_ref, b2_ref, w3_ref, b3_ref, out_ref):
    h = h1_ref[...] * st_ref[0:1, :] + st_ref[1:2, :]
    h = jnp.dot(h, w2_ref[...], preferred_element_type=jnp.float32)
    h = jnp.maximum(h + b2_ref[...], 0.0)
    out_ref[...] = (
        jnp.dot(h, w3_ref[...], preferred_element_type=jnp.float32)
        + b3_ref[...]
    )


def _mlp(emb, W1, b1, gamma, beta, W2, b2, W3, b3):
    h1, stats = pl.pallas_call(
        _stage1_body,
        grid=(NBLK,),
        in_specs=[
            pl.BlockSpec((BLK, NE), lambda i: (i, 0)),
            pl.BlockSpec((NE, 128), lambda i: (0, 0)),
            pl.BlockSpec((1, 128), lambda i: (0, 0)),
            pl.BlockSpec((1, 128), lambda i: (0, 0)),
            pl.BlockSpec((1, 128), lambda i: (0, 0)),
        ],
        out_specs=[
            pl.BlockSpec((BLK, 128), lambda i: (i, 0)),
            pl.BlockSpec((2, 128), lambda i: (0, 0)),
        ],
        out_shape=[
            jax.ShapeDtypeStruct((B, 128), jnp.float32),
            jax.ShapeDtypeStruct((2, 128), jnp.float32),
        ],
        scratch_shapes=[
            pltpu.VMEM((1, 128), jnp.float32),
            pltpu.VMEM((1, 128), jnp.float32),
        ],
    )(emb, W1, b1.reshape(1, 128), gamma.reshape(1, 128), beta.reshape(1, 128))

    out = pl.pallas_call(
        _stage2_body,
        grid=(NBLK,),
        in_specs=[
            pl.BlockSpec((BLK, 128), lambda i: (i, 0)),
            pl.BlockSpec((2, 128), lambda i: (0, 0)),
            pl.BlockSpec((128, 64), lambda i: (0, 0)),
            pl.BlockSpec((1, 64), lambda i: (0, 0)),
            pl.BlockSpec((64, 1), lambda i: (0, 0)),
            pl.BlockSpec((1, 1), lambda i: (0, 0)),
        ],
        out_specs=pl.BlockSpec((BLK, 1), lambda i: (i, 0)),
        out_shape=jax.ShapeDtypeStruct((B, 1), jnp.float32),
    )(h1, stats, W2, b2.reshape(1, 64), W3, b3.reshape(1, 1))
    return out


def kernel(x, tables, W1, b1, gamma, beta, W2, b2, W3, b3):
    offs = (jnp.arange(F, dtype=jnp.int32) * V)[None, :]
    idx2d = (x + offs).reshape((B * F) // CHUNK, CHUNK)
    tab_flat = tables.reshape(F * V, D)
    emb = _sc_gather(idx2d, tab_flat).reshape(B, NE)
    return _mlp(emb, W1, b1, gamma, beta, W2, b2, W3, b3)
